# baseline (speedup 1.0000x reference)
import functools

import jax
import jax.numpy as jnp
from jax import lax
from jax.experimental import pallas as pl
from jax.experimental.pallas import tpu as pltpu
from jax.experimental.pallas import tpu_sc as plsc

NUM_EXPERTS = 32
HIDDEN = 2880
TOP_K = 4
N_TOKENS = 8192
N_WORKERS = 32
CHUNKS = ((0, 192, 384), (6144, 64, 512))


def _mm_body(x_ref, w_ref, b_ref, out_ref, *, tpw, block_t):
    x = x_ref[...]
    w = w_ref[...]
    lg = jax.lax.dot_general(
        w, x, (((1,), (1,)), ((), ())), preferred_element_type=jnp.float32)
    lg = lg + b_ref[...]
    for j in range(block_t // tpw):
        out_ref[j] = lg[:, j * tpw:(j + 1) * tpw]


def _tc_logits(hidden_states, weight, b2, start, tpw, block_t):
    blocks = N_WORKERS * tpw // block_t
    first = start // block_t
    return pl.pallas_call(
        functools.partial(_mm_body, tpw=tpw, block_t=block_t),
        grid=(blocks,),
        in_specs=[
            pl.BlockSpec((block_t, HIDDEN), lambda i, f=first: (i + f, 0)),
            pl.BlockSpec((NUM_EXPERTS, HIDDEN), lambda i: (0, 0)),
            pl.BlockSpec((NUM_EXPERTS, 1), lambda i: (0, 0)),
        ],
        out_specs=pl.BlockSpec(
            (block_t // tpw, NUM_EXPERTS, tpw), lambda i: (i, 0, 0)),
        out_shape=jax.ShapeDtypeStruct((N_WORKERS, NUM_EXPERTS, tpw),
                                       jnp.float32),
    )(hidden_states, weight, b2)


def _make_sc_route(start, tpw):
    groups = tpw // 16

    @functools.partial(
        pl.kernel,
        mesh=plsc.VectorSubcoreMesh(core_axis_name="c", subcore_axis_name="s"),
        out_type=(),
        scratch_types=[
            pltpu.VMEM((NUM_EXPERTS * tpw,), jnp.float32),
            pltpu.VMEM((tpw, NUM_EXPERTS), jnp.float32),
            pltpu.VMEM((tpw, TOP_K), jnp.int32),
            pltpu.SemaphoreType.DMA,
        ],
        compiler_params=pltpu.CompilerParams(needs_layout_passes=False),
    )
    def _sc_route(l3_hbm, scores_ref, sel_ref, lbuf, scorebuf, selbuf, sem):
        wid = lax.axis_index("s") * 2 + lax.axis_index("c")
        base = start + wid * tpw
        cp = pltpu.async_copy(
            l3_hbm.at[pl.ds(wid * NUM_EXPERTS * tpw, NUM_EXPERTS * tpw)],
            lbuf, sem)
        iota16 = lax.iota(jnp.int32, 16)
        zero16f = jnp.zeros((16,), jnp.float32)

        def zero_body(i, carry):
            scorebuf[i, pl.ds(0, 16)] = zero16f
            scorebuf[i, pl.ds(16, 16)] = zero16f
            return carry

        lax.fori_loop(0, tpw, zero_body, 0)
        cp.wait()

        def group_body(g, carry):
            row0 = g * 16
            neg = jnp.full((16,), -jnp.inf, jnp.float32)
            zi = jnp.zeros((16,), jnp.int32)
            v0 = lbuf[pl.ds(row0, 16)]
            i0 = zi
            v1, v2, v3 = neg, neg, neg
            i1, i2, i3 = zi, zi, zi
            for e in range(1, NUM_EXPERTS):
                x = lbuf[pl.ds(e * tpw + row0, 16)]
                ei = jnp.full((16,), e, jnp.int32)
                m0 = x > v0
                m1 = x > v1
                m2 = x > v2
                m3 = x > v3
                v0, v1, v2, v3, i0, i1, i2, i3 = (
                    jnp.where(m0, x, v0),
                    jnp.where(m0, v0, jnp.where(m1, x, v1)),
                    jnp.where(m1, v1, jnp.where(m2, x, v2)),
                    jnp.where(m2, v2, jnp.where(m3, x, v3)),
                    jnp.where(m0, ei, i0),
                    jnp.where(m0, i0, jnp.where(m1, ei, i1)),
                    jnp.where(m1, i1, jnp.where(m2, ei, i2)),
                    jnp.where(m2, i2, jnp.where(m3, ei, i3)),
                )
            e1 = jnp.exp(v1 - v0)
            e2 = jnp.exp(v2 - v0)
            e3 = jnp.exp(v3 - v0)
            r = 1.0 / (1.0 + e1 + e2 + e3)
            rows = row0 + iota16
            plsc.store_scatter(scorebuf, [rows, i0], r)
            plsc.store_scatter(scorebuf, [rows, i1], e1 * r)
            plsc.store_scatter(scorebuf, [rows, i2], e2 * r)
            plsc.store_scatter(scorebuf, [rows, i3], e3 * r)
            plsc.store_scatter(selbuf, [rows, zi], i0)
            plsc.store_scatter(selbuf, [rows, zi + 1], i1)
            plsc.store_scatter(selbuf, [rows, zi + 2], i2)
            plsc.store_scatter(selbuf, [rows, zi + 3], i3)
            return carry

        lax.fori_loop(0, groups, group_body, 0)
        pltpu.sync_copy(scorebuf, scores_ref.at[pl.ds(base, tpw)])
        pltpu.sync_copy(selbuf, sel_ref.at[pl.ds(base, tpw)])

    return _sc_route


_SC_ROUTES = [_make_sc_route(start, tpw) for start, tpw, _ in CHUNKS]


@jax.jit
def kernel(hidden_states, weight, bias):
    b2 = bias.reshape(NUM_EXPERTS, 1)
    scores_ref = jax.new_ref(
        jax.lax.empty((N_TOKENS, NUM_EXPERTS), jnp.float32))
    sel_ref = jax.new_ref(jax.lax.empty((N_TOKENS, TOP_K), jnp.int32))
    for (start, tpw, block_t), sc_route in zip(CHUNKS, _SC_ROUTES):
        l3 = _tc_logits(hidden_states, weight, b2, start, tpw, block_t)
        sc_route(l3.reshape(-1), scores_ref, sel_ref)
    return scores_ref[...], sel_ref[...]


# speedup vs baseline: 1.0539x; 1.0539x over previous
import functools

import jax
import jax.numpy as jnp
from jax import lax
from jax.experimental import pallas as pl
from jax.experimental.pallas import tpu as pltpu
from jax.experimental.pallas import tpu_sc as plsc

NUM_EXPERTS = 32
HIDDEN = 2880
TOP_K = 4
N_TOKENS = 8192
N_WORKERS = 32
N_CHUNKS = 2
CHUNK = N_TOKENS // N_CHUNKS
TPW = CHUNK // N_WORKERS
GROUPS = TPW // 16
BLOCK_T = 512


def _mm_body(x_ref, w_ref, b_ref, out_ref):
    x = x_ref[...]
    w = w_ref[...]
    lg = jax.lax.dot_general(
        w, x, (((1,), (1,)), ((), ())), preferred_element_type=jnp.float32)
    lg = lg + b_ref[...]
    for j in range(BLOCK_T // TPW):
        out_ref[j] = lg[:, j * TPW:(j + 1) * TPW]


def _tc_logits(hidden_states, weight, b2, chunk):
    blocks = CHUNK // BLOCK_T
    return pl.pallas_call(
        _mm_body,
        grid=(blocks,),
        in_specs=[
            pl.BlockSpec((BLOCK_T, HIDDEN),
                         lambda i, c=chunk, nb=blocks: (i + c * nb, 0)),
            pl.BlockSpec((NUM_EXPERTS, HIDDEN), lambda i: (0, 0)),
            pl.BlockSpec((NUM_EXPERTS, 1), lambda i: (0, 0)),
        ],
        out_specs=pl.BlockSpec(
            (BLOCK_T // TPW, NUM_EXPERTS, TPW),
            lambda i: (i, 0, 0)),
        out_shape=jax.ShapeDtypeStruct((N_WORKERS, NUM_EXPERTS, TPW),
                                       jnp.float32),
    )(hidden_states, weight, b2)


def _make_sc_route(chunk):
    @functools.partial(
        pl.kernel,
        mesh=plsc.VectorSubcoreMesh(core_axis_name="c", subcore_axis_name="s"),
        out_type=(),
        scratch_types=[
            pltpu.VMEM((NUM_EXPERTS * TPW,), jnp.float32),
            pltpu.VMEM((TPW, NUM_EXPERTS), jnp.float32),
            pltpu.VMEM((TPW, TOP_K), jnp.int32),
            pltpu.SemaphoreType.DMA,
        ],
        compiler_params=pltpu.CompilerParams(needs_layout_passes=False),
    )
    def _sc_route(l3_hbm, scores_ref, sel_ref, lbuf, scorebuf, selbuf, sem):
        wid = lax.axis_index("s") * 2 + lax.axis_index("c")
        base = chunk * CHUNK + wid * TPW
        cp = pltpu.async_copy(
            l3_hbm.at[pl.ds(wid * NUM_EXPERTS * TPW, NUM_EXPERTS * TPW)],
            lbuf, sem)
        iota16 = lax.iota(jnp.int32, 16)
        zero16f = jnp.zeros((16,), jnp.float32)

        def zero_body(i, carry):
            scorebuf[i, pl.ds(0, 16)] = zero16f
            scorebuf[i, pl.ds(16, 16)] = zero16f
            return carry

        lax.fori_loop(0, TPW, zero_body, 0)
        cp.wait()

        def group_body(g, carry):
            row0 = g * 16
            neg = jnp.full((16,), -jnp.inf, jnp.float32)
            zi = jnp.zeros((16,), jnp.int32)
            v0 = lbuf[pl.ds(row0, 16)]
            i0 = zi
            v1, v2, v3 = neg, neg, neg
            i1, i2, i3 = zi, zi, zi
            for e in range(1, NUM_EXPERTS):
                x = lbuf[pl.ds(e * TPW + row0, 16)]
                ei = jnp.full((16,), e, jnp.int32)
                m0 = x > v0
                m1 = x > v1
                m2 = x > v2
                m3 = x > v3
                v0, v1, v2, v3, i0, i1, i2, i3 = (
                    jnp.where(m0, x, v0),
                    jnp.where(m0, v0, jnp.where(m1, x, v1)),
                    jnp.where(m1, v1, jnp.where(m2, x, v2)),
                    jnp.where(m2, v2, jnp.where(m3, x, v3)),
                    jnp.where(m0, ei, i0),
                    jnp.where(m0, i0, jnp.where(m1, ei, i1)),
                    jnp.where(m1, i1, jnp.where(m2, ei, i2)),
                    jnp.where(m2, i2, jnp.where(m3, ei, i3)),
                )
            e1 = jnp.exp(v1 - v0)
            e2 = jnp.exp(v2 - v0)
            e3 = jnp.exp(v3 - v0)
            r = 1.0 / (1.0 + e1 + e2 + e3)
            rows = row0 + iota16
            plsc.store_scatter(scorebuf, [rows, i0], r)
            plsc.store_scatter(scorebuf, [rows, i1], e1 * r)
            plsc.store_scatter(scorebuf, [rows, i2], e2 * r)
            plsc.store_scatter(scorebuf, [rows, i3], e3 * r)
            plsc.store_scatter(selbuf, [rows, zi], i0)
            plsc.store_scatter(selbuf, [rows, zi + 1], i1)
            plsc.store_scatter(selbuf, [rows, zi + 2], i2)
            plsc.store_scatter(selbuf, [rows, zi + 3], i3)
            return carry

        lax.fori_loop(0, GROUPS, group_body, 0)
        pltpu.sync_copy(scorebuf, scores_ref.at[pl.ds(base, TPW)])
        pltpu.sync_copy(selbuf, sel_ref.at[pl.ds(base, TPW)])

    return _sc_route


_SC_ROUTES = [_make_sc_route(c) for c in range(N_CHUNKS)]


@jax.jit
def kernel(hidden_states, weight, bias):
    b2 = bias.reshape(NUM_EXPERTS, 1)
    scores_ref = jax.new_ref(
        jax.lax.empty((N_TOKENS, NUM_EXPERTS), jnp.float32))
    sel_ref = jax.new_ref(jax.lax.empty((N_TOKENS, TOP_K), jnp.int32))
    for c in range(N_CHUNKS):
        l3 = _tc_logits(hidden_states, weight, b2, c)
        _SC_ROUTES[c](l3.reshape(-1), scores_ref, sel_ref)
    return scores_ref[...], sel_ref[...]
